# baseline (speedup 1.0000x reference)
import functools

import jax
import jax.numpy as jnp
from jax import lax
from jax.experimental import pallas as pl
from jax.experimental.pallas import tpu as pltpu
from jax.experimental.pallas import tpu_sc as plsc

N = 50000
E_N = 800000
DV = 133
DE = 14
HID = 64
NG = 1024
HH = 256

NC = 2
NS = 16
NW = NC * NS
F32 = jnp.float32

E_PAD = 800768
N_PAD = 50176
NPACK = N_PAD // 2
SEG_RANGE = 15360
SEG_TROWS = 7808
GM_HALF = NG // 2
GM_TROWS = 640

IW = 8
W = IW * 128


def _mesh():
    return plsc.VectorSubcoreMesh(core_axis_name="c", subcore_axis_name="s")


def _sc_gather(tabs, idxs2d):
    n = len(tabs)
    dw = 256
    nsub = W // dw
    nrows = idxs2d[0].shape[0] * 128
    nchunks = nrows // W
    kmax = -(-nchunks // NW)
    scratch = [pltpu.VMEM((IW, 128), jnp.int32) for _ in range(n)]
    scratch += [pltpu.VMEM((dw, 128), F32), pltpu.VMEM((dw, 128), F32),
                pltpu.SemaphoreType.DMA, pltpu.SemaphoreType.DMA,
                pltpu.SemaphoreType.DMA, pltpu.SemaphoreType.DMA]

    @functools.partial(
        pl.kernel, mesh=_mesh(),
        out_type=[jax.ShapeDtypeStruct((nrows, 128), F32) for _ in range(n)],
        scratch_types=scratch)
    def k(*refs):
        tab_refs = refs[:n]
        idx_refs = refs[n:2 * n]
        out_refs = refs[2 * n:3 * n]
        ibs = refs[3 * n:4 * n]
        rb = refs[4 * n:4 * n + 2]
        gsem = refs[4 * n + 2:4 * n + 4]
        wsem = refs[4 * n + 4:4 * n + 6]
        wid = lax.axis_index("s") * NC + lax.axis_index("c")

        @pl.loop(0, kmax)
        def _(kk):
            c = wid + kk * NW

            @pl.when(c < nchunks)
            def _():
                for t in range(n):
                    pltpu.sync_copy(idx_refs[t].at[pl.ds(c * IW, IW)], ibs[t])
                writes = [None, None]
                for i in range(n * nsub):
                    t, p = divmod(i, nsub)
                    b = i % 2
                    if writes[b] is not None:
                        writes[b].wait()
                    copies = []
                    for j in range(dw // 128):
                        copies.append(pltpu.make_async_copy(
                            tab_refs[t].at[ibs[t].at[p * (dw // 128) + j]],
                            rb[b].at[pl.ds(j * 128, 128)], gsem[b]))
                        copies[-1].start()
                    for cp in copies:
                        cp.wait()
                    writes[b] = pltpu.make_async_copy(
                        rb[b], out_refs[t].at[pl.ds(c * W + p * dw, dw)],
                        wsem[b])
                    writes[b].start()
                for wcp in writes:
                    if wcp is not None:
                        wcp.wait()

    return k(*tabs, *idxs2d)


def _sc_segsum(x, idx2d, zeros_tab, out_rows, half, shift, trows, drows,
               passes, dps):
    thalf = half >> shift
    nchunks = x.shape[0] // W
    assert nchunks * W == x.shape[0]
    kmax = -(-nchunks // NS)
    zr = trows // NS
    assert thalf % drows == 0 and thalf % 8 == 0 and drows % 8 == 0
    assert trows >= thalf + NS * dps
    dk = -(-(thalf // drows) // NS)

    @functools.partial(
        pl.kernel, mesh=_mesh(),
        out_type=jax.ShapeDtypeStruct((out_rows, 128), F32),
        scratch_types=[pltpu.VMEM((IW, 128), jnp.int32),
                       pltpu.VMEM((IW, 128), jnp.int32),
                       pltpu.VMEM((W // 4, 128), F32),
                       pltpu.VMEM((W // 4, 128), F32),
                       pltpu.SemaphoreType.DMA, pltpu.SemaphoreType.DMA,
                       pltpu.VMEM_SHARED((trows, 128), F32)])
    def k(x_hbm, idx_hbm, z_hbm, out_hbm, ib, ib2, xb0, xb1, sem0, sem1,
          table):
        core = lax.axis_index("c")
        sid = lax.axis_index("s")
        for ps in range(passes):
            rng = core + 2 * ps
            base = rng * half
            pltpu.sync_copy(z_hbm.at[pl.ds(sid * zr, zr)],
                            table.at[pl.ds(sid * zr, zr)])
            plsc.subcore_barrier()

            @pl.loop(0, kmax)
            def _(kk):
                c = sid + kk * NS

                @pl.when(c < nchunks)
                def _():
                    xbs = [xb0, xb1]
                    sems = [sem0, sem1]
                    qw = W // 4
                    loads = [None, None]
                    loads[0] = pltpu.make_async_copy(
                        x_hbm.at[pl.ds(c * W, qw)], xb0, sem0)
                    loads[0].start()
                    pltpu.sync_copy(idx_hbm.at[pl.ds(c * IW, IW)], ib)
                    for j in range(IW):
                        @pl.loop(0, 128, step=16)
                        def _(l):
                            v = ib[j, pl.ds(l, 16)]
                            inr = (v >= base) & (v < base + half)
                            dump = (thalf + sid * dps
                                    + (lax.iota(jnp.int32, 16) & (dps - 1)))
                            ib2[j, pl.ds(l, 16)] = jnp.where(
                                inr, (v - base) >> shift, dump)
                    for p in range(4):
                        if p < 3:
                            b = (p + 1) % 2
                            loads[b] = pltpu.make_async_copy(
                                x_hbm.at[pl.ds(c * W + (p + 1) * qw, qw)],
                                xbs[b], sems[b])
                            loads[b].start()
                        loads[p % 2].wait()
                        for j in range(qw // 128):
                            pltpu.sync_copy(
                                xbs[p % 2].at[pl.ds(j * 128, 128)],
                                table.at[ib2.at[p * (qw // 128) + j]],
                                add=True)

            plsc.subcore_barrier()
            nd = jnp.minimum(thalf, out_rows - rng * thalf) // drows

            @pl.loop(0, dk)
            def _(kk):
                c = sid + kk * NS

                @pl.when(c < nd)
                def _():
                    pltpu.sync_copy(
                        table.at[pl.ds(c * drows, drows)],
                        out_hbm.at[pl.ds(rng * thalf + c * drows, drows)])

            plsc.subcore_barrier()

    return k(x, idx2d, zeros_tab)


def _place(h, par_col):
    b = h.shape[0]
    z = jnp.zeros((b, HID), F32)
    lo = jnp.concatenate([h, z], axis=1)
    hi = jnp.concatenate([z, h], axis=1)
    return jnp.where((par_col & 1) == 1, hi, lo)


def _node_proj(V, Wi_v, Wo_v):
    BN = 2000

    def body(v, wi, wo, o):
        vv = v[...]
        o[...] = jnp.concatenate(
            [jnp.dot(vv, wi[...], preferred_element_type=F32),
             jnp.dot(vv, wo[...], preferred_element_type=F32)], axis=1)

    return pl.pallas_call(
        body,
        grid=(N // BN,),
        in_specs=[pl.BlockSpec((BN, DV), lambda i: (i, 0)),
                  pl.BlockSpec((DV, HID), lambda i: (0, 0)),
                  pl.BlockSpec((DV, HID), lambda i: (0, 0))],
        out_specs=pl.BlockSpec((BN, 2 * HID), lambda i: (i, 0)),
        out_shape=jax.ShapeDtypeStruct((N, 2 * HID), F32),
    )(V, Wi_v, Wo_v)


def _edge_init(Psrc, E, Wi_e, dstc):
    BE = 2048

    def body(p, e, w, d, o):
        h = jnp.maximum(
            p[:, :HID] + jnp.dot(e[...], w[...], preferred_element_type=F32),
            0.0)
        o[...] = _place(h, d[...])

    return pl.pallas_call(
        body,
        grid=(E_PAD // BE,),
        in_specs=[pl.BlockSpec((BE, 128), lambda i: (i, 0)),
                  pl.BlockSpec((BE, DE), lambda i: (i, 0)),
                  pl.BlockSpec((DE, HID), lambda i: (0, 0)),
                  pl.BlockSpec((BE, 1), lambda i: (i, 0))],
        out_specs=pl.BlockSpec((BE, 128), lambda i: (i, 0)),
        out_shape=jax.ShapeDtypeStruct((E_PAD, 128), F32),
    )(Psrc, E, Wi_e, dstc)


def _msg_update_half(Gh, Rh, H0P, W_h, dstc, srcc, off, prev):
    BE = 1024
    nb = Gh.shape[0] // BE

    def body(g, r, h0, w, d, s, *rest):
        o = rest[-1]
        gv = g[...]
        gsel = jnp.where((s[...] & 1) == 1, gv[:, HID:], gv[:, :HID])
        m = gsel - (r[:, :HID] + r[:, HID:])
        h0v = h0[:, :HID] + h0[:, HID:]
        h = jnp.maximum(
            h0v + jnp.dot(m, w[...], preferred_element_type=F32), 0.0)
        o[...] = _place(h, d[...])

    in_specs = [pl.BlockSpec((BE, 128), lambda i: (i, 0)),
                pl.BlockSpec((BE, 128), lambda i: (i, 0)),
                pl.BlockSpec((BE, 128), lambda i: (i + off, 0)),
                pl.BlockSpec((HID, HID), lambda i: (0, 0)),
                pl.BlockSpec((BE, 1), lambda i: (i + off, 0)),
                pl.BlockSpec((BE, 1), lambda i: (i + off, 0))]
    args = [Gh, Rh, H0P, W_h, dstc, srcc]
    kwargs = {}
    if prev is not None:
        in_specs.append(pl.BlockSpec((8, 128), lambda i: (0, 0)))
        args.append(prev)
        kwargs["input_output_aliases"] = {6: 0}
    return pl.pallas_call(
        body,
        grid=(nb,),
        in_specs=in_specs,
        out_specs=pl.BlockSpec((BE, 128), lambda i: (i + off, 0)),
        out_shape=jax.ShapeDtypeStruct((E_PAD, 128), F32),
        **kwargs,
    )(*args)


def _node_final(P128, Mvg, Wo_m, b_o2):
    BN = 3136

    def body(pq, mvp, w, b, o):
        par = jax.lax.broadcasted_iota(jnp.int32, (BN, 1), 0) & 1
        mvv = mvp[...]
        mv = jnp.where(par == 1, mvv[:, HID:], mvv[:, :HID])
        h = jnp.maximum(
            pq[:, HID:]
            + jnp.dot(mv, w[...], preferred_element_type=F32)
            + b[...], 0.0)
        ones = jnp.ones((BN, 1), F32)
        zer = jnp.zeros((BN, 128 - HID - 1), F32)
        o[...] = jnp.concatenate([h, ones, zer], axis=1)

    return pl.pallas_call(
        body,
        grid=(N_PAD // BN,),
        in_specs=[pl.BlockSpec((BN, 128), lambda i: (i, 0)),
                  pl.BlockSpec((BN, 128), lambda i: (i, 0)),
                  pl.BlockSpec((HID, HID), lambda i: (0, 0)),
                  pl.BlockSpec((1, HID), lambda i: (0, 0))],
        out_specs=pl.BlockSpec((BN, 128), lambda i: (i, 0)),
        out_shape=jax.ShapeDtypeStruct((N_PAD, 128), F32),
    )(P128, Mvg, Wo_m, b_o2)


def _head(T, W1, b1r, W2, b2r):
    def body(t, w1, b1, w2, b2, o):
        tv = t[...]
        zz = tv[:, :HID] / jnp.maximum(tv[:, HID:HID + 1], 1.0)
        a = jnp.maximum(
            jnp.dot(zz, w1[...], preferred_element_type=F32) + b1[...], 0.0)
        o[...] = jnp.dot(a, w2[...], preferred_element_type=F32) + b2[...]

    return pl.pallas_call(
        body,
        out_shape=jax.ShapeDtypeStruct((NG, 1), F32),
    )(T, W1, b1r, W2, b2r)


def kernel(V, E, W_i, W_h, W_o, b_o, W1, b1, W2, b2,
           edge_index, rev_edge_index, batch):
    src = edge_index[0]
    dst = edge_index[1]
    Wi_v, Wi_e = W_i[:DV], W_i[DV:]
    Wo_v, Wo_m = W_o[:DV], W_o[DV:]

    srcp = jnp.pad(src, (0, E_PAD - E_N))
    src2d = srcp.reshape(E_PAD // 128, 128)
    srch2d = (srcp >> 1).reshape(E_PAD // 128, 128)
    srcc = srcp.reshape(E_PAD, 1)
    unp2d = (jnp.arange(N_PAD, dtype=jnp.int32) >> 1).reshape(
        N_PAD // 128, 128)
    rev2d = jnp.pad(rev_edge_index, (0, E_PAD - E_N)
                    ).reshape(E_PAD // 128, 128)
    dstp = jnp.pad(dst, (0, E_PAD - E_N), constant_values=N_PAD)
    dst2d = dstp.reshape(E_PAD // 128, 128)
    dstc = dstp.reshape(E_PAD, 1)
    batch2d = jnp.pad(batch, (0, N_PAD - N),
                      constant_values=NG).reshape(N_PAD // 128, 128)

    zt = jnp.zeros((SEG_TROWS, 128), F32)
    ztm = jnp.zeros((GM_TROWS, 128), F32)

    P128 = _node_proj(V, Wi_v, Wo_v)
    (Psrc,) = _sc_gather([P128], [src2d])
    HtP = _edge_init(Psrc, E, Wi_e, dstc)
    H0P = HtP

    for _ in range(2):
        Mvp = _sc_segsum(HtP, dst2d, zt, NPACK, SEG_RANGE, 1, SEG_TROWS,
                           128, 2, 8)
        hrows = E_PAD // 256
        Ga, Ra = _sc_gather([Mvp, HtP], [srch2d[:hrows], rev2d[:hrows]])
        Gb, Rb = _sc_gather([Mvp, HtP], [srch2d[hrows:], rev2d[hrows:]])
        Hta = _msg_update_half(Ga, Ra, H0P, W_h, dstc, srcc, 0, None)
        HtP = _msg_update_half(Gb, Rb, H0P, W_h, dstc, srcc,
                               E_PAD // 2048, Hta)

    Mvp = _sc_segsum(HtP, dst2d, zt, NPACK, SEG_RANGE, 1, SEG_TROWS,
                       128, 2, 8)
    (Mvg,) = _sc_gather([Mvp], [unp2d])
    Hvp = _node_final(P128, Mvg, Wo_m, b_o.reshape(1, HID))
    T = _sc_segsum(Hvp, batch2d, ztm, NG, GM_HALF, 0, GM_TROWS, 32, 1, 8)

    return _head(T, W1, b1.reshape(1, HH), W2, b2.reshape(1, 1))


# speedup vs baseline: 1.0261x; 1.0261x over previous
import functools

import jax
import jax.numpy as jnp
from jax import lax
from jax.experimental import pallas as pl
from jax.experimental.pallas import tpu as pltpu
from jax.experimental.pallas import tpu_sc as plsc

N = 50000
E_N = 800000
DV = 133
DE = 14
HID = 64
NG = 1024
HH = 256

NC = 2
NS = 16
NW = NC * NS
F32 = jnp.float32

E_PAD = 800768
N_PAD = 50176
NPACK = N_PAD // 2
SEG_RANGE = 15360
SEG_TROWS = 7808
GM_HALF = NG // 2
GM_TROWS = 640

IW = 8
W = IW * 128


def _mesh():
    return plsc.VectorSubcoreMesh(core_axis_name="c", subcore_axis_name="s")


def _sc_gather(tabs, idxs2d):
    n = len(tabs)
    dw = 256
    nsub = W // dw
    nrows = idxs2d[0].shape[0] * 128
    nchunks = nrows // W
    kmax = -(-nchunks // NW)
    scratch = [pltpu.VMEM((IW, 128), jnp.int32) for _ in range(n)]
    scratch += [pltpu.VMEM((dw, 128), F32), pltpu.VMEM((dw, 128), F32),
                pltpu.SemaphoreType.DMA, pltpu.SemaphoreType.DMA,
                pltpu.SemaphoreType.DMA, pltpu.SemaphoreType.DMA]

    @functools.partial(
        pl.kernel, mesh=_mesh(),
        out_type=[jax.ShapeDtypeStruct((nrows, 128), F32) for _ in range(n)],
        scratch_types=scratch)
    def k(*refs):
        tab_refs = refs[:n]
        idx_refs = refs[n:2 * n]
        out_refs = refs[2 * n:3 * n]
        ibs = refs[3 * n:4 * n]
        rb = refs[4 * n:4 * n + 2]
        gsem = refs[4 * n + 2:4 * n + 4]
        wsem = refs[4 * n + 4:4 * n + 6]
        wid = lax.axis_index("s") * NC + lax.axis_index("c")

        @pl.loop(0, kmax)
        def _(kk):
            c = wid + kk * NW

            @pl.when(c < nchunks)
            def _():
                for t in range(n):
                    pltpu.sync_copy(idx_refs[t].at[pl.ds(c * IW, IW)], ibs[t])
                writes = [None, None]
                for i in range(n * nsub):
                    t, p = divmod(i, nsub)
                    b = i % 2
                    if writes[b] is not None:
                        writes[b].wait()
                    copies = []
                    for j in range(dw // 128):
                        copies.append(pltpu.make_async_copy(
                            tab_refs[t].at[ibs[t].at[p * (dw // 128) + j]],
                            rb[b].at[pl.ds(j * 128, 128)], gsem[b]))
                        copies[-1].start()
                    for cp in copies:
                        cp.wait()
                    writes[b] = pltpu.make_async_copy(
                        rb[b], out_refs[t].at[pl.ds(c * W + p * dw, dw)],
                        wsem[b])
                    writes[b].start()
                for wcp in writes:
                    if wcp is not None:
                        wcp.wait()

    return k(*tabs, *idxs2d)


def _sc_segsum(x, idx2d, zeros_tab, out_rows, half, shift, trows, drows,
               passes, dps):
    thalf = half >> shift
    nchunks = x.shape[0] // W
    assert nchunks * W == x.shape[0]
    kmax = -(-nchunks // NS)
    zr = trows // NS
    assert thalf % drows == 0 and thalf % 8 == 0 and drows % 8 == 0
    assert trows >= thalf + NS * dps
    dk = -(-(thalf // drows) // NS)

    @functools.partial(
        pl.kernel, mesh=_mesh(),
        out_type=jax.ShapeDtypeStruct((out_rows, 128), F32),
        scratch_types=[pltpu.VMEM((IW, 128), jnp.int32),
                       pltpu.VMEM((IW, 128), jnp.int32),
                       pltpu.VMEM((W // 4, 128), F32),
                       pltpu.VMEM((W // 4, 128), F32),
                       pltpu.SemaphoreType.DMA, pltpu.SemaphoreType.DMA,
                       pltpu.VMEM_SHARED((trows, 128), F32)])
    def k(x_hbm, idx_hbm, z_hbm, out_hbm, ib, ib2, xb0, xb1, sem0, sem1,
          table):
        core = lax.axis_index("c")
        sid = lax.axis_index("s")
        for ps in range(passes):
            rng = core + 2 * ps
            base = rng * half
            pltpu.sync_copy(z_hbm.at[pl.ds(sid * zr, zr)],
                            table.at[pl.ds(sid * zr, zr)])
            plsc.subcore_barrier()

            @pl.loop(0, kmax)
            def _(kk):
                c = sid + kk * NS

                @pl.when(c < nchunks)
                def _():
                    xbs = [xb0, xb1]
                    sems = [sem0, sem1]
                    qw = W // 4
                    loads = [None, None]
                    loads[0] = pltpu.make_async_copy(
                        x_hbm.at[pl.ds(c * W, qw)], xb0, sem0)
                    loads[0].start()
                    pltpu.sync_copy(idx_hbm.at[pl.ds(c * IW, IW)], ib)
                    for j in range(IW):
                        @pl.loop(0, 128, step=16)
                        def _(l):
                            v = ib[j, pl.ds(l, 16)]
                            inr = (v >= base) & (v < base + half)
                            dump = (thalf + sid * dps
                                    + (lax.iota(jnp.int32, 16) & (dps - 1)))
                            ib2[j, pl.ds(l, 16)] = jnp.where(
                                inr, (v - base) >> shift, dump)
                    for p in range(4):
                        if p < 3:
                            b = (p + 1) % 2
                            loads[b] = pltpu.make_async_copy(
                                x_hbm.at[pl.ds(c * W + (p + 1) * qw, qw)],
                                xbs[b], sems[b])
                            loads[b].start()
                        loads[p % 2].wait()
                        for j in range(qw // 128):
                            pltpu.sync_copy(
                                xbs[p % 2].at[pl.ds(j * 128, 128)],
                                table.at[ib2.at[p * (qw // 128) + j]],
                                add=True)

            plsc.subcore_barrier()
            nd = jnp.minimum(thalf, out_rows - rng * thalf) // drows

            @pl.loop(0, dk)
            def _(kk):
                c = sid + kk * NS

                @pl.when(c < nd)
                def _():
                    pltpu.sync_copy(
                        table.at[pl.ds(c * drows, drows)],
                        out_hbm.at[pl.ds(rng * thalf + c * drows, drows)])

            plsc.subcore_barrier()

    return k(x, idx2d, zeros_tab)


def _place(h, par_col):
    b = h.shape[0]
    z = jnp.zeros((b, HID), F32)
    lo = jnp.concatenate([h, z], axis=1)
    hi = jnp.concatenate([z, h], axis=1)
    return jnp.where((par_col & 1) == 1, hi, lo)


def _node_proj(V, Wi_v, Wo_v):
    BN = 2000

    def body(v, wi, wo, o):
        vv = v[...]
        o[...] = jnp.concatenate(
            [jnp.dot(vv, wi[...], preferred_element_type=F32),
             jnp.dot(vv, wo[...], preferred_element_type=F32)], axis=1)

    return pl.pallas_call(
        body,
        grid=(N // BN,),
        in_specs=[pl.BlockSpec((BN, DV), lambda i: (i, 0)),
                  pl.BlockSpec((DV, HID), lambda i: (0, 0)),
                  pl.BlockSpec((DV, HID), lambda i: (0, 0))],
        out_specs=pl.BlockSpec((BN, 2 * HID), lambda i: (i, 0)),
        out_shape=jax.ShapeDtypeStruct((N, 2 * HID), F32),
    )(V, Wi_v, Wo_v)


def _edge_init(Psrc, E, Wi_e, dstc):
    BE = 2048

    def body(p, e, w, d, o):
        h = jnp.maximum(
            p[:, :HID] + jnp.dot(e[...], w[...], preferred_element_type=F32),
            0.0)
        o[...] = _place(h, d[...])

    return pl.pallas_call(
        body,
        grid=(E_PAD // BE,),
        in_specs=[pl.BlockSpec((BE, 128), lambda i: (i, 0)),
                  pl.BlockSpec((BE, DE), lambda i: (i, 0)),
                  pl.BlockSpec((DE, HID), lambda i: (0, 0)),
                  pl.BlockSpec((BE, 1), lambda i: (i, 0))],
        out_specs=pl.BlockSpec((BE, 128), lambda i: (i, 0)),
        out_shape=jax.ShapeDtypeStruct((E_PAD, 128), F32),
    )(Psrc, E, Wi_e, dstc)


def _msg_update(Gt, Rt, H0P, W_h, dstc, srcc):
    BE = 2048

    def body(g, r, h0, w, d, s, o):
        gv = g[...]
        gsel = jnp.where((s[...] & 1) == 1, gv[:, HID:], gv[:, :HID])
        m = gsel - (r[:, :HID] + r[:, HID:])
        h0v = h0[:, :HID] + h0[:, HID:]
        h = jnp.maximum(
            h0v + jnp.dot(m, w[...], preferred_element_type=F32), 0.0)
        o[...] = _place(h, d[...])

    return pl.pallas_call(
        body,
        grid=(E_PAD // BE,),
        in_specs=[pl.BlockSpec((BE, 128), lambda i: (i, 0)),
                  pl.BlockSpec((BE, 128), lambda i: (i, 0)),
                  pl.BlockSpec((BE, 128), lambda i: (i, 0)),
                  pl.BlockSpec((HID, HID), lambda i: (0, 0)),
                  pl.BlockSpec((BE, 1), lambda i: (i, 0)),
                  pl.BlockSpec((BE, 1), lambda i: (i, 0))],
        out_specs=pl.BlockSpec((BE, 128), lambda i: (i, 0)),
        out_shape=jax.ShapeDtypeStruct((E_PAD, 128), F32),
    )(Gt, Rt, H0P, W_h, dstc, srcc)


def _node_final(P128, Mvg, Wo_m, b_o2):
    BN = 3136

    def body(pq, mvp, w, b, o):
        par = jax.lax.broadcasted_iota(jnp.int32, (BN, 1), 0) & 1
        mvv = mvp[...]
        mv = jnp.where(par == 1, mvv[:, HID:], mvv[:, :HID])
        h = jnp.maximum(
            pq[:, HID:]
            + jnp.dot(mv, w[...], preferred_element_type=F32)
            + b[...], 0.0)
        ones = jnp.ones((BN, 1), F32)
        zer = jnp.zeros((BN, 128 - HID - 1), F32)
        o[...] = jnp.concatenate([h, ones, zer], axis=1)

    return pl.pallas_call(
        body,
        grid=(N_PAD // BN,),
        in_specs=[pl.BlockSpec((BN, 128), lambda i: (i, 0)),
                  pl.BlockSpec((BN, 128), lambda i: (i, 0)),
                  pl.BlockSpec((HID, HID), lambda i: (0, 0)),
                  pl.BlockSpec((1, HID), lambda i: (0, 0))],
        out_specs=pl.BlockSpec((BN, 128), lambda i: (i, 0)),
        out_shape=jax.ShapeDtypeStruct((N_PAD, 128), F32),
    )(P128, Mvg, Wo_m, b_o2)


def _head(T, W1, b1r, W2, b2r):
    def body(t, w1, b1, w2, b2, o):
        tv = t[...]
        zz = tv[:, :HID] / jnp.maximum(tv[:, HID:HID + 1], 1.0)
        a = jnp.maximum(
            jnp.dot(zz, w1[...], preferred_element_type=F32) + b1[...], 0.0)
        o[...] = jnp.dot(a, w2[...], preferred_element_type=F32) + b2[...]

    return pl.pallas_call(
        body,
        out_shape=jax.ShapeDtypeStruct((NG, 1), F32),
    )(T, W1, b1r, W2, b2r)


def kernel(V, E, W_i, W_h, W_o, b_o, W1, b1, W2, b2,
           edge_index, rev_edge_index, batch):
    src = edge_index[0]
    dst = edge_index[1]
    Wi_v, Wi_e = W_i[:DV], W_i[DV:]
    Wo_v, Wo_m = W_o[:DV], W_o[DV:]

    srcp = jnp.pad(src, (0, E_PAD - E_N))
    src2d = srcp.reshape(E_PAD // 128, 128)
    srch2d = (srcp >> 1).reshape(E_PAD // 128, 128)
    srcc = srcp.reshape(E_PAD, 1)
    unp2d = (jnp.arange(N_PAD, dtype=jnp.int32) >> 1).reshape(
        N_PAD // 128, 128)
    rev2d = jnp.pad(rev_edge_index, (0, E_PAD - E_N)
                    ).reshape(E_PAD // 128, 128)
    dstp = jnp.pad(dst, (0, E_PAD - E_N), constant_values=N_PAD)
    dst2d = dstp.reshape(E_PAD // 128, 128)
    dstc = dstp.reshape(E_PAD, 1)
    batch2d = jnp.pad(batch, (0, N_PAD - N),
                      constant_values=NG).reshape(N_PAD // 128, 128)

    zt = jnp.zeros((SEG_TROWS, 128), F32)
    ztm = jnp.zeros((GM_TROWS, 128), F32)

    P128 = _node_proj(V, Wi_v, Wo_v)
    (Psrc,) = _sc_gather([P128], [src2d])
    HtP = _edge_init(Psrc, E, Wi_e, dstc)
    H0P = HtP

    for _ in range(2):
        Mvp = _sc_segsum(HtP, dst2d, zt, NPACK, SEG_RANGE, 1, SEG_TROWS,
                           128, 2, 8)
        Gt, Rt = _sc_gather([Mvp, HtP], [srch2d, rev2d])
        HtP = _msg_update(Gt, Rt, H0P, W_h, dstc, srcc)

    Mvp = _sc_segsum(HtP, dst2d, zt, NPACK, SEG_RANGE, 1, SEG_TROWS,
                       128, 2, 8)
    (Mvg,) = _sc_gather([Mvp], [unp2d])
    Hvp = _node_final(P128, Mvg, Wo_m, b_o.reshape(1, HID))
    T = _sc_segsum(Hvp, batch2d, ztm, NG, GM_HALF, 0, GM_TROWS, 32, 1, 8)

    return _head(T, W1, b1.reshape(1, HH), W2, b2.reshape(1, 1))
